# baseline (speedup 1.0000x reference)
import functools

import jax
import jax.numpy as jnp
from jax import lax
from jax.experimental import pallas as pl
from jax.experimental.pallas import tpu as pltpu
from jax.experimental.pallas import tpu_sc as plsc

_SCALE = 3.1622776601683795

_NUM_WORKERS = 32
_CHUNK = 128
_VBLK = 512


def _prep_body(in_ref, out_ref):
    D = in_ref.shape[0]
    eye = jnp.eye(D, dtype=jnp.float32)
    t = lax.dot_general(in_ref[...] * _SCALE, eye, (((0,), (0,)), ((), ())),
                        preferred_element_type=jnp.float32)
    t3 = jnp.reshape(t, (t.shape[0] // 2, 2, D))
    out_ref[:, 0:D] = t3[:, 0, :]
    out_ref[:, D:2 * D] = t3[:, 1, :]


def _table_prep(table_t):
    D, V = table_t.shape
    grid = (V + _VBLK - 1) // _VBLK
    return pl.pallas_call(
        _prep_body,
        grid=(grid,),
        in_specs=[pl.BlockSpec((D, _VBLK), lambda i: (0, i))],
        out_specs=pl.BlockSpec((_VBLK // 2, 2 * D), lambda i: (i, 0)),
        out_shape=jax.ShapeDtypeStruct((V // 2, 2 * D), jnp.float32),
    )(table_t)


def _finish_body(in_ref, out_ref):
    D = out_ref.shape[1] * 8
    n = in_ref.shape[0]
    eye = jnp.eye(n, dtype=jnp.float32)
    t = lax.dot_general(in_ref[:, 0:D], eye, (((0,), (0,)), ((), ())),
                        preferred_element_type=jnp.float32)
    out_ref[...] = jnp.reshape(t, out_ref.shape)


def _finish(rows, H, B, D):
    nb = B // _CHUNK
    return pl.pallas_call(
        _finish_body,
        grid=(H, nb),
        in_specs=[pl.BlockSpec((_CHUNK, _CHUNK), lambda h, c: (h * nb + c, 0))],
        out_specs=pl.BlockSpec((1, D // 8, 1, 8, _CHUNK),
                               lambda h, c: (h, 0, c, 0, 0)),
        out_shape=jax.ShapeDtypeStruct((H, D // 8, nb, 8, _CHUNK),
                                       jnp.float32),
    )(rows)


def _emb_call(n_chunks, D, N):
    mesh = plsc.VectorSubcoreMesh(core_axis_name="c", subcore_axis_name="s")

    @functools.partial(
        pl.kernel,
        mesh=mesh,
        out_type=jax.ShapeDtypeStruct((N, _CHUNK), jnp.float32),
        scratch_types=(
            [pltpu.VMEM((n_chunks, _CHUNK), jnp.int32)]
            + [pltpu.VMEM((_CHUNK, D), jnp.float32) for _ in range(4)]
            + [pltpu.SemaphoreType.DMA for _ in range(8)]
        ),
        compiler_params=pltpu.CompilerParams(use_tc_tiling_on_sc=False),
    )
    def emb(idx_hbm, table_hbm, out_hbm, idx_v,
            b0, b1, b2, b3, g0, g1, g2, g3, s0, s1, s2, s3):
        bufs = (b0, b1, b2, b3)
        gs = (g0, g1, g2, g3)
        ss = (s0, s1, s2, s3)
        wid = lax.axis_index("s") * 2 + lax.axis_index("c")
        crow = wid * n_chunks
        pltpu.sync_copy(idx_hbm.at[pl.ds(crow, n_chunks)], idx_v)

        def gather_start(j, b):
            pltpu.async_copy(table_hbm.at[idx_v.at[j]], bufs[b], gs[b])

        def gather_wait(j, b):
            pltpu.make_async_copy(table_hbm.at[idx_v.at[j]], bufs[b], gs[b]).wait()

        def store_start(j, b):
            pltpu.async_copy(
                bufs[b],
                out_hbm.at[pl.ds((crow + j) * _CHUNK, _CHUNK), pl.ds(0, D)],
                ss[b])

        def store_wait(b):
            pltpu.make_async_copy(
                bufs[b],
                out_hbm.at[pl.ds(crow * _CHUNK, _CHUNK), pl.ds(0, D)],
                ss[b]).wait()

        gather_start(0, 0)
        gather_start(1, 1)
        gather_start(2, 2)
        gather_wait(0, 0)
        store_start(0, 0)
        gather_start(3, 3)
        gather_wait(1, 1)
        store_start(1, 1)

        def step(jj, carry):
            j0 = 2 + jj * 4
            for t in range(4):
                j = j0 + t
                b = (2 + t) % 4
                bg = t % 4
                store_wait(bg)
                gather_start(j + 2, bg)
                gather_wait(j, b)
                store_start(j, b)
            return carry

        lax.fori_loop(0, (n_chunks - 4) // 4, step, 0)

        gather_wait(n_chunks - 2, 2)
        store_start(n_chunks - 2, 2)
        gather_wait(n_chunks - 1, 3)
        store_start(n_chunks - 1, 3)
        for b in range(4):
            store_wait(b)

    return emb


def kernel(x, table):
    B, H = x.shape
    V, D = table.shape
    N = B * H
    assert N % (_NUM_WORKERS * _CHUNK) == 0 and D % 16 == 0 and V % 2 == 0
    n_chunks = N // (_NUM_WORKERS * _CHUNK)
    assert n_chunks % 4 == 0 and n_chunks >= 8
    xt = jnp.transpose(x.astype(jnp.int32))
    idx = xt.reshape(N // _CHUNK, _CHUNK)
    t2 = _table_prep(jnp.transpose(table))
    t64 = jnp.reshape(t2, (V, D))
    rows = _emb_call(n_chunks, D, N)(idx, t64)
    out5 = _finish(rows, H, B, D)
    out = jnp.transpose(
        jnp.reshape(jnp.transpose(out5, (0, 1, 3, 2, 4)), (H, D, B)),
        (2, 0, 1))
    return out


# speedup vs baseline: 4.1380x; 4.1380x over previous
import functools

import jax
import jax.numpy as jnp
from jax import lax
from jax.experimental import pallas as pl
from jax.experimental.pallas import tpu as pltpu
from jax.experimental.pallas import tpu_sc as plsc

_SCALE = 3.1622776601683795

_NUM_WORKERS = 32
_CHUNK = 128
_ROWS_PER_IT = 8


def _emb_call(n_chunks, D, N):
    mesh = plsc.VectorSubcoreMesh(core_axis_name="c", subcore_axis_name="s")

    @functools.partial(
        pl.kernel,
        mesh=mesh,
        out_type=jax.ShapeDtypeStruct((N, D), jnp.float32),
        scratch_types=(
            [pltpu.VMEM((n_chunks, _CHUNK), jnp.int32)]
            + [pltpu.VMEM((_CHUNK, D), jnp.float32) for _ in range(4)]
            + [pltpu.SemaphoreType.DMA for _ in range(8)]
        ),
        compiler_params=pltpu.CompilerParams(use_tc_tiling_on_sc=False),
    )
    def emb(idx_hbm, table_hbm, out_hbm, idx_v,
            b0, b1, b2, b3, g0, g1, g2, g3, s0, s1, s2, s3):
        bufs = (b0, b1, b2, b3)
        gs = (g0, g1, g2, g3)
        ss = (s0, s1, s2, s3)
        wid = lax.axis_index("s") * 2 + lax.axis_index("c")
        crow = wid * n_chunks
        pltpu.sync_copy(idx_hbm.at[pl.ds(crow, n_chunks)], idx_v)

        def gather_start(j, b):
            pltpu.async_copy(table_hbm.at[idx_v.at[j]], bufs[b], gs[b])

        def gather_wait(j, b):
            pltpu.make_async_copy(table_hbm.at[idx_v.at[j]], bufs[b], gs[b]).wait()

        def store_start(j, b):
            pltpu.async_copy(bufs[b], out_hbm.at[pl.ds((crow + j) * _CHUNK, _CHUNK)], ss[b])

        def store_wait(b):
            pltpu.make_async_copy(bufs[b], out_hbm.at[pl.ds(crow * _CHUNK, _CHUNK)], ss[b]).wait()

        def scale(b):
            buf = bufs[b]

            def body(i, carry):
                r0 = i * _ROWS_PER_IT
                for rr in range(_ROWS_PER_IT):
                    for c in range(D // 16):
                        buf[r0 + rr, pl.ds(c * 16, 16)] = (
                            buf[r0 + rr, pl.ds(c * 16, 16)] * _SCALE)
                return carry

            lax.fori_loop(0, _CHUNK // _ROWS_PER_IT, body, 0)

        gather_start(0, 0)
        gather_start(1, 1)
        gather_start(2, 2)
        gather_wait(0, 0)
        scale(0)
        store_start(0, 0)
        gather_start(3, 3)
        gather_wait(1, 1)
        scale(1)
        store_start(1, 1)

        def step(jj, carry):
            j0 = 2 + jj * 4
            for t in range(4):
                j = j0 + t
                b = (2 + t) % 4
                bg = t % 4
                store_wait(bg)
                gather_start(j + 2, bg)
                gather_wait(j, b)
                scale(b)
                store_start(j, b)
            return carry

        lax.fori_loop(0, (n_chunks - 4) // 4, step, 0)

        gather_wait(n_chunks - 2, 2)
        scale(2)
        store_start(n_chunks - 2, 2)
        gather_wait(n_chunks - 1, 3)
        scale(3)
        store_start(n_chunks - 1, 3)
        for b in range(4):
            store_wait(b)

    return emb


def kernel(x, table):
    B, H = x.shape
    V, D = table.shape
    N = B * H
    assert N % (_NUM_WORKERS * _CHUNK) == 0 and D % 16 == 0
    n_chunks = N // (_NUM_WORKERS * _CHUNK)
    assert n_chunks % 4 == 0 and n_chunks >= 8
    idx = x.reshape(_NUM_WORKERS * n_chunks, _CHUNK).astype(jnp.int32)
    out = _emb_call(n_chunks, D, N)(idx, table)
    return out.reshape(B, H, D)
